# Initial kernel scaffold; baseline (speedup 1.0000x reference)
import jax
import jax.numpy as jnp
from jax.experimental import pallas as pl


def kernel(x, gemm_edges, W, b):
    raise NotImplementedError("write your pallas kernel here")


# speedup vs baseline: 5.3310x; 5.3310x over previous
import functools

import jax
import jax.numpy as jnp
from jax import lax
from jax.experimental import pallas as pl
from jax.experimental.pallas import tpu as pltpu
from jax.experimental.pallas import tpu_sc as plsc

_NC = 2
_NS = 16
_NW = _NC * _NS


def _sc_gather(xt, idx_flat, E, C, K):
    e_per_w = E // _NW
    nchunks = e_per_w // K
    mesh = plsc.VectorSubcoreMesh(core_axis_name="c", subcore_axis_name="s")

    @functools.partial(
        pl.kernel,
        mesh=mesh,
        out_type=jax.ShapeDtypeStruct((4 * E, C), jnp.float32),
        scratch_types=[
            pltpu.VMEM((K,), jnp.int32),
            pltpu.VMEM((K,), jnp.int32),
            pltpu.VMEM((K,), jnp.int32),
            pltpu.VMEM((K,), jnp.int32),
            pltpu.VMEM((K, C), jnp.float32),
            pltpu.VMEM((K, C), jnp.float32),
            pltpu.VMEM((K, C), jnp.float32),
            pltpu.VMEM((K, C), jnp.float32),
            pltpu.SemaphoreType.DMA,
        ],
    )
    def gather_kernel(xt_hbm, idx_hbm, out_hbm, i0, i1, i2, i3,
                      r0, r1, r2, r3, sem):
        wid = lax.axis_index("s") * _NC + lax.axis_index("c")
        w_base = wid * e_per_w
        idxv = (i0, i1, i2, i3)
        rows = (r0, r1, r2, r3)

        def body(c, carry):
            base = pl.multiple_of(w_base + c * K, 8)
            for j in range(4):
                pltpu.sync_copy(idx_hbm.at[pl.ds(j * E + base, K)], idxv[j])
            cps = [pltpu.async_copy(xt_hbm.at[idxv[j]], rows[j], sem)
                   for j in range(4)]
            for cp in cps:
                cp.wait()
            for j in range(4):
                pltpu.sync_copy(rows[j], out_hbm.at[pl.ds(j * E + base, K)])
            return carry

        lax.fori_loop(0, nchunks, body, 0)

    return gather_kernel(xt, idx_flat)


def _tc_combine_conv(xt, raw, wcat, bias, E, C, Eb):
    def body(xt_ref, raw_ref, w_ref, b_ref, out_ref):
        f0 = xt_ref[...]
        g1 = raw_ref[0]
        g2 = raw_ref[1]
        g3 = raw_ref[2]
        g4 = raw_ref[3]
        feat = jnp.concatenate(
            [f0, g1 + g3, g2 + g4, jnp.abs(g1 - g3), jnp.abs(g2 - g4)],
            axis=-1)
        out_ref[...] = (
            jnp.dot(feat, w_ref[...], preferred_element_type=jnp.float32)
            + b_ref[...])

    return pl.pallas_call(
        body,
        grid=(E // Eb,),
        in_specs=[
            pl.BlockSpec((Eb, C), lambda i: (i, 0)),
            pl.BlockSpec((4, Eb, C), lambda i: (0, i, 0)),
            pl.BlockSpec((5 * C, 32), lambda i: (0, 0)),
            pl.BlockSpec((1, 32), lambda i: (0, 0)),
        ],
        out_specs=pl.BlockSpec((Eb, 32), lambda i: (i, 0)),
        out_shape=jax.ShapeDtypeStruct((E, 32), jnp.float32),
    )(xt, raw, wcat, bias)


def kernel(x, gemm_edges, W, b):
    Bq, C, E = x.shape
    xt = jnp.transpose(x[0])
    idx_flat = jnp.transpose(gemm_edges[0]).reshape(-1)

    raw = _sc_gather(xt, idx_flat, E, C, K=200)
    raw = raw.reshape(4, E, C)

    w5 = W[:, :, 0, :]
    wcat = jnp.transpose(w5, (2, 1, 0)).reshape(5 * C, 32)
    out = _tc_combine_conv(xt, raw, wcat, b.reshape(1, 32), E, C, Eb=1600)
    return jnp.transpose(out)[None, :, :, None]
